# baseline (speedup 1.0000x reference)
import functools

import jax
import jax.numpy as jnp
from jax import lax
from jax.experimental import pallas as pl
from jax.experimental.pallas import tpu as pltpu
from jax.experimental.pallas import tpu_sc as plsc

PRE_NMS_THRESH = 0.05
PRE_NMS_TOP_N = 1000
NMS_THRESH = 0.6
FPN_POST_NMS_TOP_N = 100
MIN_SIZE = 0.0
N, C, H, W = 2, 80, 100, 152
HW = H * W
CHW = C * HW
TOPP = 1024
CAND = 1536
SCOUT = 4096
DUMP = 3072
NTILE = 16
PERTILE = CHW // NTILE
ONE_BITS = 0x3F800000

_HI = lax.Precision.HIGHEST


def _comb_bisect_body(cls_ref, ctr_ref, comb_ref, thr_ref):
    s = jax.nn.sigmoid(cls_ref[...])
    c = jax.nn.sigmoid(ctr_ref[...])
    comb = jnp.where(s > PRE_NMS_THRESH, s * c, 0.0)
    comb_ref[...] = comb
    ci = lax.bitcast_convert_type(comb, jnp.int32)

    def body(_, lohi):
        lo, hi = lohi
        mid = (lo + hi) >> 1
        cnt = jnp.sum((ci > mid).astype(jnp.int32))
        pred = cnt >= PRE_NMS_TOP_N
        return jnp.where(pred, mid, lo), jnp.where(pred, hi, mid)

    lo, _ = lax.fori_loop(0, 31, body, (jnp.int32(-1), jnp.int32(ONE_BITS)))
    thr = lax.bitcast_convert_type(jnp.maximum(lo, 0), jnp.float32)
    thr_ref[...] = jnp.broadcast_to(thr, (1, 1, 128))


def _comb_scores_bisect(box_cls, centerness):
    cls3 = box_cls.reshape(N, C, HW)
    ctr3 = centerness.reshape(N, 1, HW)
    return pl.pallas_call(
        _comb_bisect_body,
        grid=(N,),
        in_specs=[
            pl.BlockSpec((1, C, HW), lambda i: (i, 0, 0)),
            pl.BlockSpec((1, 1, HW), lambda i: (i, 0, 0)),
        ],
        out_specs=[
            pl.BlockSpec((1, C, HW), lambda i: (i, 0, 0)),
            pl.BlockSpec((1, 1, 128), lambda i: (i, 0, 0)),
        ],
        out_shape=[
            jax.ShapeDtypeStruct((N, C, HW), jnp.float32),
            jax.ShapeDtypeStruct((N, 1, 128), jnp.float32),
        ],
    )(cls3, ctr3)


def _sc_body(comb_hbm, thr_hbm, tab_hbm, rows_hbm,
             data_v, thr_v, vals_c, idx_c, pos_c, loc_c, rows_v, off_sm,
             sem, sem2):
    img = lax.axis_index("c")
    sub = lax.axis_index("s")
    base = sub * PERTILE
    pltpu.sync_copy(comb_hbm.at[pl.ds(img * CHW + base, PERTILE)], data_v)
    pltpu.sync_copy(thr_hbm.at[pl.ds(img * 16, 16)], thr_v)
    thr = thr_v[...]
    lane = lax.iota(jnp.int32, 16)

    def fill_body(q, _):
        rows_v[0, q >> 3, pl.ds((q & 7) * 16, 16)] = jnp.full((16,), -1.0, jnp.float32)
        return 0

    lax.fori_loop(0, 1024, fill_body, 0)
    seg = SCOUT // NTILE
    pltpu.sync_copy(rows_v.at[0], rows_hbm.at[pl.ds(img * SCOUT + sub * seg, 128)])
    pltpu.sync_copy(rows_v.at[0], rows_hbm.at[pl.ds(img * SCOUT + sub * seg + 128, 128)])

    BLK = 128

    def blk_body(ib, carry):
        cnt, cs = carry
        base_e = ib * BLK
        tv = jnp.zeros((16,), jnp.int32)
        for u in range(BLK // 16):
            v = data_v[pl.ds(base_e + u * 16, 16)]
            tv = tv + jnp.where(v > thr, 1, 0)
        t = jnp.sum(tv)

        @pl.when(t > 0)
        def _():
            c = cnt
            for u in range(BLK // 16):
                v = data_v[pl.ds(base_e + u * 16, 16)]
                m = v > thr
                mi = jnp.where(m, 1, 0)
                pos = c + plsc.cumsum(mi) - mi
                ok = m & (pos < CAND)
                plsc.store_scatter(vals_c, [pos >> 7, pos & 127], v, mask=ok)
                gi = lane + (base_e + u * 16 + base)
                plsc.store_scatter(idx_c, [pos >> 7, pos & 127], gi, mask=ok)
                c = c + jnp.sum(mi)

        return cnt + t, cs + t

    cnt, cs = jnp.zeros((16,), jnp.int32), jnp.int32(0)

    off_sm[0] = 0
    plsc.subcore_barrier()
    for j in range(NTILE):
        plsc.fetch_and_add(off_sm, jnp.where(j > sub, cs, 0), subcore_id=j)
    plsc.subcore_barrier()
    off = jnp.zeros((16,), jnp.int32) + off_sm[0]

    def pos_body(q, _):
        r = lane + q * 16
        posg = img * SCOUT + jnp.where((r < cnt) & ((off + r) < CAND), off + r, DUMP)
        pos_c[q >> 3, pl.ds((q & 7) * 16, 16)] = posg
        gi = idx_c[q >> 3, pl.ds((q & 7) * 16, 16)]
        loc = gi - (gi // HW) * HW
        loc = jnp.clip(loc, 0, HW - 1) + img * HW
        loc_c[q >> 3, pl.ds((q & 7) * 16, 16)] = loc
        return 0

    lax.fori_loop(0, 128, pos_body, 0)

    for j in range(16):
        b = j % 2
        g = pltpu.async_copy(tab_hbm.at[loc_c.at[j]], rows_v.at[b], sem2)
        g.wait()
        for t in range(8):
            rt = lane + t * 16
            v16 = vals_c[j, pl.ds(t * 16, 16)]
            i16 = idx_c[j, pl.ds(t * 16, 16)]
            plsc.store_scatter(rows_v.at[b], [rt, jnp.full((16,), 6, jnp.int32)], v16)
            plsc.store_scatter(rows_v.at[b], [rt, jnp.full((16,), 7, jnp.int32)],
                               i16.astype(jnp.float32))
        sc = pltpu.async_copy(rows_v.at[b], rows_hbm.at[pos_c.at[j]], sem2)
        sc.wait()


@functools.lru_cache(maxsize=1)
def _sc_compact_kernel():
    @functools.partial(
        pl.kernel,
        mesh=plsc.VectorSubcoreMesh(core_axis_name="c", subcore_axis_name="s"),
        compiler_params=pltpu.CompilerParams(needs_layout_passes=False),
        out_type=[
            jax.ShapeDtypeStruct((N * SCOUT, 128), jnp.float32),
        ],
        scratch_types=[
            pltpu.VMEM((PERTILE,), jnp.float32),
            pltpu.VMEM((16,), jnp.float32),
            pltpu.VMEM((16, 128), jnp.float32),
            pltpu.VMEM((16, 128), jnp.int32),
            pltpu.VMEM((16, 128), jnp.int32),
            pltpu.VMEM((16, 128), jnp.int32),
            pltpu.VMEM((2, 128, 128), jnp.float32),
            pltpu.SMEM((1,), jnp.int32),
            pltpu.SemaphoreType.DMA,
            pltpu.SemaphoreType.DMA,
        ],
    )
    def _sc_compact(comb_hbm, thr_hbm, tab_hbm, rows_hbm, *scratch):
        _sc_body(comb_hbm, thr_hbm, tab_hbm, rows_hbm, *scratch)

    return _sc_compact


def _sort_gather_body(vr_ref, vc_ref, ir_ref, ic_ref, rows_ref, rowsT_ref,
                      sz_ref, rd_ref, cd_ref):
    f32 = jnp.float32
    vrow = vr_ref[0]
    vcol = vc_ref[0]
    irow = ir_ref[0]
    icol = ic_ref[0]
    h_img = sz_ref[0, 0, 0]
    w_img = sz_ref[0, 0, 1]

    ranks = []
    rrow = jnp.zeros((1, CAND), f32)
    for sl in range(CAND // 128):
        vi = vcol[sl * 128:(sl + 1) * 128]
        ii = icol[sl * 128:(sl + 1) * 128]
        cmp = ((vrow > vi) | ((vrow == vi) & (irow < ii))).astype(f32)
        ranks.append(jnp.sum(cmp, axis=1, keepdims=True))
        rrow = rrow + jnp.sum(cmp, axis=0, keepdims=True)
    rank_col = jnp.concatenate(ranks, axis=0)
    rank_row = (CAND - 1.0) - rrow

    PT = (rank_col.astype(jnp.int32)
          == lax.broadcasted_iota(jnp.int32, (CAND, TOPP), 1)).astype(f32)
    P = (rank_row.astype(jnp.int32)
         == lax.broadcasted_iota(jnp.int32, (TOPP, CAND), 0)).astype(f32)
    accT = jnp.dot(rowsT_ref[0], PT, preferred_element_type=f32, precision=_HI)
    accR = jnp.dot(P, rows_ref[0], preferred_element_type=f32, precision=_HI)
    s_val = accT[6:7]
    s_idxf = accT[7:8]
    s_idxf_c = accR[:, 7:8]

    lane = lax.broadcasted_iota(jnp.int32, (1, TOPP), 1)
    ts = jnp.where(lane < PRE_NMS_TOP_N, s_val, -1.0)
    sidx_r = s_idxf.astype(jnp.int32)
    sidx_c = s_idxf_c.astype(jnp.int32)
    cls_row = ((sidx_r // HW) + 1).astype(f32)
    cls_col = ((sidx_c // HW) + 1).astype(f32)

    r0, r1, r2, r3 = (accT[0:1], accT[1:2], accT[2:3], accT[3:4])
    px, py = accT[4:5], accT[5:6]
    x1 = jnp.clip(px - r0, 0.0, w_img - 1.0)
    y1 = jnp.clip(py - r1, 0.0, h_img - 1.0)
    x2 = jnp.clip(px + r2, 0.0, w_img - 1.0)
    y2 = jnp.clip(py + r3, 0.0, h_img - 1.0)
    rd_ref[0, 0:1, :] = ts
    rd_ref[0, 1:2, :] = cls_row
    rd_ref[0, 2:3, :] = x1
    rd_ref[0, 3:4, :] = y1
    rd_ref[0, 4:5, :] = x2
    rd_ref[0, 5:6, :] = y2
    rd_ref[0, 6:7, :] = jnp.zeros((1, TOPP), f32)
    rd_ref[0, 7:8, :] = jnp.zeros((1, TOPP), f32)

    c0, c1, c2, c3 = (accR[:, 0:1], accR[:, 1:2], accR[:, 2:3], accR[:, 3:4])
    pxc, pyc = accR[:, 4:5], accR[:, 5:6]
    x1c = jnp.clip(pxc - c0, 0.0, w_img - 1.0)
    y1c = jnp.clip(pyc - c1, 0.0, h_img - 1.0)
    x2c = jnp.clip(pxc + c2, 0.0, w_img - 1.0)
    y2c = jnp.clip(pyc + c3, 0.0, h_img - 1.0)
    zc = jnp.zeros((TOPP, 1), f32)
    cd_ref[0, :, 0:1] = x1c
    cd_ref[0, :, 1:2] = y1c
    cd_ref[0, :, 2:3] = x2c
    cd_ref[0, :, 3:4] = y2c
    cd_ref[0, :, 4:5] = cls_col
    cd_ref[0, :, 5:6] = zc
    cd_ref[0, :, 6:7] = zc
    cd_ref[0, :, 7:8] = zc


def _sort_gather_call(vals, idxf, rows, rowsT, sizes_f):
    spec3 = lambda a, b: pl.BlockSpec((1, a, b), lambda i: (i, 0, 0))
    rowdat, coldat = pl.pallas_call(
        _sort_gather_body,
        grid=(N,),
        in_specs=[
            spec3(1, CAND),
            spec3(CAND, 1),
            spec3(1, CAND),
            spec3(CAND, 1),
            spec3(CAND, 16),
            spec3(16, CAND),
            spec3(1, 8),
        ],
        out_specs=[spec3(8, TOPP), spec3(TOPP, 8)],
        out_shape=[
            jax.ShapeDtypeStruct((N, 8, TOPP), jnp.float32),
            jax.ShapeDtypeStruct((N, TOPP, 8), jnp.float32),
        ],
    )(vals.reshape(N, 1, CAND), vals.reshape(N, CAND, 1),
      idxf.reshape(N, 1, CAND), idxf.reshape(N, CAND, 1),
      rows, rowsT, sizes_f)
    return rowdat, coldat


def _nms_body(rd_ref, cd_ref, bx_ref, sc_ref, lb_ref, adj_ref):
    f32 = jnp.float32
    ts = rd_ref[0, 0:1, :]
    cls_row = rd_ref[0, 1:2, :]
    x1 = rd_ref[0, 2:3, :]
    y1 = rd_ref[0, 3:4, :]
    x2 = rd_ref[0, 4:5, :]
    y2 = rd_ref[0, 5:6, :]

    off_row = cls_row * 10000.0
    bnx1, bny1, bnx2, bny2 = x1 + off_row, y1 + off_row, x2 + off_row, y2 + off_row
    area_row = (bnx2 - bnx1) * (bny2 - bny1)

    det = jnp.sqrt(jnp.clip(ts, 1e-12, None))
    valid = (ts > 0) & ((x2 - x1) >= MIN_SIZE) & ((y2 - y1) >= MIN_SIZE)

    x1c = cd_ref[0, :, 0:1]
    y1c = cd_ref[0, :, 1:2]
    x2c = cd_ref[0, :, 2:3]
    y2c = cd_ref[0, :, 3:4]
    off_col = cd_ref[0, :, 4:5] * 10000.0
    bnx1c, bny1c = x1c + off_col, y1c + off_col
    bnx2c, bny2c = x2c + off_col, y2c + off_col
    area_col = (bnx2c - bnx1c) * (bny2c - bny1c)

    SLAB = 128
    iota_j = lax.broadcasted_iota(jnp.int32, (SLAB, TOPP), 1)
    for s in range(TOPP // SLAB):
        r = slice(s * SLAB, (s + 1) * SLAB)
        xx1 = jnp.maximum(bnx1c[r], bnx1)
        yy1 = jnp.maximum(bny1c[r], bny1)
        xx2 = jnp.minimum(bnx2c[r], bnx2)
        yy2 = jnp.minimum(bny2c[r], bny2)
        iw = jnp.clip(xx2 - xx1, 0.0, None)
        ih = jnp.clip(yy2 - yy1, 0.0, None)
        inter = iw * ih
        iou = inter / jnp.maximum(area_col[r] + area_row - inter, 1e-8)
        tri = iota_j > (s * SLAB + lax.broadcasted_iota(jnp.int32, (SLAB, TOPP), 0))
        adj_ref[r, :] = jnp.where((iou > NMS_THRESH) & tri, 1.0, 0.0)

    lane = lax.broadcasted_iota(jnp.int32, (1, TOPP), 1)
    sup0 = jnp.where(valid, 0.0, 1.0)

    def scan_body(i, sup):
        row = adj_ref[pl.ds(i, 1), :]
        sup_i = jnp.sum(jnp.where(lane == i, sup, 0.0))
        return jnp.maximum(sup, row * (1.0 - jnp.minimum(sup_i, 1.0)))

    sup = lax.fori_loop(0, PRE_NMS_TOP_N, scan_body, sup0)

    final = jnp.where(valid & (sup < 0.5), det, -1.0)

    out_iota = lax.broadcasted_iota(jnp.int32, (1, 128), 1)

    def pick_body(t, carry):
        fin, s_row, l_row, ox1, oy1, ox2, oy2 = carry
        m = jnp.max(fin)
        sel = jnp.min(jnp.where(fin == m, lane, TOPP + 1))
        selm = lane == sel
        ok = m > 0

        def e(v):
            return jnp.sum(jnp.where(selm, v, 0.0))

        tm = out_iota == t
        s_row = jnp.where(tm, jnp.where(ok, m, 0.0), s_row)
        l_row = jnp.where(tm, jnp.where(ok, e(cls_row), 0.0), l_row)
        ox1 = jnp.where(tm, jnp.where(ok, e(x1), 0.0), ox1)
        oy1 = jnp.where(tm, jnp.where(ok, e(y1), 0.0), oy1)
        ox2 = jnp.where(tm, jnp.where(ok, e(x2), 0.0), ox2)
        oy2 = jnp.where(tm, jnp.where(ok, e(y2), 0.0), oy2)
        fin = jnp.where(selm, -2.0, fin)
        return fin, s_row, l_row, ox1, oy1, ox2, oy2

    z = jnp.zeros((1, 128), f32)
    carry = (final, z, z, z, z, z, z)
    _, s_row, l_row, ox1, oy1, ox2, oy2 = lax.fori_loop(
        0, FPN_POST_NMS_TOP_N, pick_body, carry)

    sc_ref[0] = s_row
    lb_ref[0] = l_row.astype(jnp.int32)
    bx_ref[0, 0:1, :] = ox1
    bx_ref[0, 1:2, :] = oy1
    bx_ref[0, 2:3, :] = ox2
    bx_ref[0, 3:4, :] = oy2


def _nms_call(rowdat, coldat):
    spec3 = lambda a, b: pl.BlockSpec((1, a, b), lambda i: (i, 0, 0))
    boxes4, scores, labels = pl.pallas_call(
        _nms_body,
        grid=(N,),
        in_specs=[spec3(8, TOPP), spec3(TOPP, 8)],
        out_specs=[spec3(4, 128), spec3(1, 128), spec3(1, 128)],
        out_shape=[
            jax.ShapeDtypeStruct((N, 4, 128), jnp.float32),
            jax.ShapeDtypeStruct((N, 1, 128), jnp.float32),
            jax.ShapeDtypeStruct((N, 1, 128), jnp.int32),
        ],
        scratch_shapes=[pltpu.VMEM((TOPP, TOPP), jnp.float32)],
    )(rowdat, coldat)
    return boxes4, scores, labels


def kernel(locations, box_cls, box_regression, ang_regression, centerness, image_sizes, is_rotated):
    comb, thr = _comb_scores_bisect(box_cls, centerness)

    regT = box_regression.reshape(N, 4, HW).transpose(0, 2, 1)
    locb = jnp.broadcast_to(locations[None], (N, HW, 2))
    tab = jnp.concatenate(
        [regT, locb, jnp.zeros((N, HW, 122), jnp.float32)], axis=2)
    tab = tab.reshape(N * HW, 128)

    (rows_sc,) = _sc_compact_kernel()(
        comb.reshape(N * CHW), thr[:, 0, :16].reshape(N * 16), tab)
    rows = rows_sc.reshape(N, SCOUT, 128)[:, :CAND, :16]
    vals = rows[:, :, 6]
    idxf = rows[:, :, 7]
    rowsT = rows.transpose(0, 2, 1)

    sizes_f = jnp.pad(image_sizes.astype(jnp.float32), ((0, 0), (0, 6)))
    sizes_f = sizes_f.reshape(N, 1, 8)

    rowdat, coldat = _sort_gather_call(vals, idxf, rows, rowsT, sizes_f)
    boxes4, scores, labels = _nms_call(rowdat, coldat)
    out_boxes = boxes4[:, :, :FPN_POST_NMS_TOP_N].transpose(0, 2, 1)
    out_scores = scores[:, 0, :FPN_POST_NMS_TOP_N]
    out_labels = labels[:, 0, :FPN_POST_NMS_TOP_N]
    return out_boxes, out_scores, out_labels


# speedup vs baseline: 2.8525x; 2.8525x over previous
import functools

import jax
import jax.numpy as jnp
from jax import lax
from jax.experimental import pallas as pl
from jax.experimental.pallas import tpu as pltpu
from jax.experimental.pallas import tpu_sc as plsc

PRE_NMS_THRESH = 0.05
PRE_NMS_TOP_N = 1000
NMS_THRESH = 0.6
FPN_POST_NMS_TOP_N = 100
MIN_SIZE = 0.0
N, C, H, W = 2, 80, 100, 152
HW = H * W
CHW = C * HW
TOPP = 1024
CAND = 1536
SCOUT = 4096
DUMP = 3072
NTILE = 16
PERTILE = CHW // NTILE
ONE_BITS = 0x3F800000

_HI = lax.Precision.HIGHEST


def _comb_bisect_body(cls_ref, ctr_ref, comb_ref, thr_ref):
    s = jax.nn.sigmoid(cls_ref[...])
    c = jax.nn.sigmoid(ctr_ref[...])
    comb = jnp.where(s > PRE_NMS_THRESH, s * c, 0.0)
    comb_ref[...] = comb
    ci = lax.bitcast_convert_type(comb, jnp.int32)

    def body(_, lohi):
        lo, hi = lohi
        mid = (lo + hi) >> 1
        cnt = jnp.sum((ci > mid).astype(jnp.int32))
        pred = cnt >= PRE_NMS_TOP_N
        return jnp.where(pred, mid, lo), jnp.where(pred, hi, mid)

    lo, _ = lax.fori_loop(0, 31, body, (jnp.int32(-1), jnp.int32(ONE_BITS)))
    thr = lax.bitcast_convert_type(jnp.maximum(lo, 0), jnp.float32)
    thr_ref[...] = jnp.broadcast_to(thr, (1, 1, 128))


def _comb_scores_bisect(box_cls, centerness):
    cls3 = box_cls.reshape(N, C, HW)
    ctr3 = centerness.reshape(N, 1, HW)
    return pl.pallas_call(
        _comb_bisect_body,
        grid=(N,),
        in_specs=[
            pl.BlockSpec((1, C, HW), lambda i: (i, 0, 0)),
            pl.BlockSpec((1, 1, HW), lambda i: (i, 0, 0)),
        ],
        out_specs=[
            pl.BlockSpec((1, C, HW), lambda i: (i, 0, 0)),
            pl.BlockSpec((1, 1, 128), lambda i: (i, 0, 0)),
        ],
        out_shape=[
            jax.ShapeDtypeStruct((N, C, HW), jnp.float32),
            jax.ShapeDtypeStruct((N, 1, 128), jnp.float32),
        ],
    )(cls3, ctr3)


def _sc_body(comb_hbm, thr_hbm, tab_hbm, rows_hbm,
             data_v, thr_v, vals_c, idx_c, pos_c, loc_c, rows_v, off_sm,
             sem, sem2):
    img = lax.axis_index("c")
    sub = lax.axis_index("s")
    base = sub * PERTILE
    pltpu.sync_copy(comb_hbm.at[pl.ds(img * CHW + base, PERTILE)], data_v)
    pltpu.sync_copy(thr_hbm.at[pl.ds(img * 16, 16)], thr_v)
    thr = thr_v[...]
    lane = lax.iota(jnp.int32, 16)

    def fill_body(q, _):
        rows_v[0, q >> 3, pl.ds((q & 7) * 16, 16)] = jnp.full((16,), -1.0, jnp.float32)
        return 0

    lax.fori_loop(0, 1024, fill_body, 0)
    seg = SCOUT // NTILE
    pltpu.sync_copy(rows_v.at[0], rows_hbm.at[pl.ds(img * SCOUT + sub * seg, 128)])
    pltpu.sync_copy(rows_v.at[0], rows_hbm.at[pl.ds(img * SCOUT + sub * seg + 128, 128)])

    BLK = 128

    def blk_body(ib, carry):
        cnt, cs = carry
        base_e = ib * BLK
        tv = jnp.zeros((16,), jnp.int32)
        for u in range(BLK // 16):
            v = data_v[pl.ds(base_e + u * 16, 16)]
            tv = tv + jnp.where(v > thr, 1, 0)
        t = jnp.sum(tv)

        @pl.when(t > 0)
        def _():
            c = cnt
            for u in range(BLK // 16):
                v = data_v[pl.ds(base_e + u * 16, 16)]
                m = v > thr
                mi = jnp.where(m, 1, 0)
                pos = c + plsc.cumsum(mi) - mi
                ok = m & (pos < CAND)
                plsc.store_scatter(vals_c, [pos >> 7, pos & 127], v, mask=ok)
                gi = lane + (base_e + u * 16 + base)
                plsc.store_scatter(idx_c, [pos >> 7, pos & 127], gi, mask=ok)
                c = c + jnp.sum(mi)

        return cnt + t, cs + t

    cnt, cs = jnp.zeros((16,), jnp.int32), jnp.int32(0)

    off_sm[0] = 0
    plsc.subcore_barrier()
    for j in range(NTILE):
        plsc.fetch_and_add(off_sm, jnp.where(j > sub, cs, 0), subcore_id=j)
    plsc.subcore_barrier()
    off = jnp.zeros((16,), jnp.int32) + off_sm[0]

    def pos_body(q, _):
        r = lane + q * 16
        posg = img * SCOUT + jnp.where((r < cnt) & ((off + r) < CAND), off + r, DUMP)
        pos_c[q >> 3, pl.ds((q & 7) * 16, 16)] = posg
        gi = idx_c[q >> 3, pl.ds((q & 7) * 16, 16)]
        loc = gi - (gi // HW) * HW
        loc = jnp.clip(loc, 0, HW - 1) + img * HW
        loc_c[q >> 3, pl.ds((q & 7) * 16, 16)] = loc
        return 0

    lax.fori_loop(0, 128, pos_body, 0)

    g = pltpu.async_copy(tab_hbm.at[loc_c.at[0]], rows_v.at[0], sem2)
    g.wait()
    sc = pltpu.async_copy(rows_v.at[0], rows_hbm.at[pos_c.at[0]], sem2)
    sc.wait()


@functools.lru_cache(maxsize=1)
def _sc_compact_kernel():
    @functools.partial(
        pl.kernel,
        mesh=plsc.VectorSubcoreMesh(core_axis_name="c", subcore_axis_name="s"),
        compiler_params=pltpu.CompilerParams(needs_layout_passes=False),
        out_type=[
            jax.ShapeDtypeStruct((N * SCOUT, 128), jnp.float32),
        ],
        scratch_types=[
            pltpu.VMEM((PERTILE,), jnp.float32),
            pltpu.VMEM((16,), jnp.float32),
            pltpu.VMEM((16, 128), jnp.float32),
            pltpu.VMEM((16, 128), jnp.int32),
            pltpu.VMEM((16, 128), jnp.int32),
            pltpu.VMEM((16, 128), jnp.int32),
            pltpu.VMEM((2, 128, 128), jnp.float32),
            pltpu.SMEM((1,), jnp.int32),
            pltpu.SemaphoreType.DMA,
            pltpu.SemaphoreType.DMA,
        ],
    )
    def _sc_compact(comb_hbm, thr_hbm, tab_hbm, rows_hbm, *scratch):
        _sc_body(comb_hbm, thr_hbm, tab_hbm, rows_hbm, *scratch)

    return _sc_compact


def _sort_gather_body(vr_ref, vc_ref, ir_ref, ic_ref, rows_ref, rowsT_ref,
                      sz_ref, rd_ref, cd_ref):
    f32 = jnp.float32
    vrow = vr_ref[0]
    vcol = vc_ref[0]
    irow = ir_ref[0]
    icol = ic_ref[0]
    h_img = sz_ref[0, 0, 0]
    w_img = sz_ref[0, 0, 1]

    ranks = []
    rrow = jnp.zeros((1, CAND), f32)
    for sl in range(CAND // 128):
        vi = vcol[sl * 128:(sl + 1) * 128]
        ii = icol[sl * 128:(sl + 1) * 128]
        cmp = ((vrow > vi) | ((vrow == vi) & (irow < ii))).astype(f32)
        ranks.append(jnp.sum(cmp, axis=1, keepdims=True))
        rrow = rrow + jnp.sum(cmp, axis=0, keepdims=True)
    rank_col = jnp.concatenate(ranks, axis=0)
    rank_row = (CAND - 1.0) - rrow

    PT = (rank_col.astype(jnp.int32)
          == lax.broadcasted_iota(jnp.int32, (CAND, TOPP), 1)).astype(f32)
    P = (rank_row.astype(jnp.int32)
         == lax.broadcasted_iota(jnp.int32, (TOPP, CAND), 0)).astype(f32)
    accT = jnp.dot(rowsT_ref[0], PT, preferred_element_type=f32, precision=_HI)
    accR = jnp.dot(P, rows_ref[0], preferred_element_type=f32, precision=_HI)
    s_val = accT[6:7]
    s_idxf = accT[7:8]
    s_idxf_c = accR[:, 7:8]

    lane = lax.broadcasted_iota(jnp.int32, (1, TOPP), 1)
    ts = jnp.where(lane < PRE_NMS_TOP_N, s_val, -1.0)
    sidx_r = s_idxf.astype(jnp.int32)
    sidx_c = s_idxf_c.astype(jnp.int32)
    cls_row = ((sidx_r // HW) + 1).astype(f32)
    cls_col = ((sidx_c // HW) + 1).astype(f32)

    r0, r1, r2, r3 = (accT[0:1], accT[1:2], accT[2:3], accT[3:4])
    px, py = accT[4:5], accT[5:6]
    x1 = jnp.clip(px - r0, 0.0, w_img - 1.0)
    y1 = jnp.clip(py - r1, 0.0, h_img - 1.0)
    x2 = jnp.clip(px + r2, 0.0, w_img - 1.0)
    y2 = jnp.clip(py + r3, 0.0, h_img - 1.0)
    rd_ref[0, 0:1, :] = ts
    rd_ref[0, 1:2, :] = cls_row
    rd_ref[0, 2:3, :] = x1
    rd_ref[0, 3:4, :] = y1
    rd_ref[0, 4:5, :] = x2
    rd_ref[0, 5:6, :] = y2
    rd_ref[0, 6:7, :] = jnp.zeros((1, TOPP), f32)
    rd_ref[0, 7:8, :] = jnp.zeros((1, TOPP), f32)

    c0, c1, c2, c3 = (accR[:, 0:1], accR[:, 1:2], accR[:, 2:3], accR[:, 3:4])
    pxc, pyc = accR[:, 4:5], accR[:, 5:6]
    x1c = jnp.clip(pxc - c0, 0.0, w_img - 1.0)
    y1c = jnp.clip(pyc - c1, 0.0, h_img - 1.0)
    x2c = jnp.clip(pxc + c2, 0.0, w_img - 1.0)
    y2c = jnp.clip(pyc + c3, 0.0, h_img - 1.0)
    zc = jnp.zeros((TOPP, 1), f32)
    cd_ref[0, :, 0:1] = x1c
    cd_ref[0, :, 1:2] = y1c
    cd_ref[0, :, 2:3] = x2c
    cd_ref[0, :, 3:4] = y2c
    cd_ref[0, :, 4:5] = cls_col
    cd_ref[0, :, 5:6] = zc
    cd_ref[0, :, 6:7] = zc
    cd_ref[0, :, 7:8] = zc


def _sort_gather_call(vals, idxf, rows, rowsT, sizes_f):
    spec3 = lambda a, b: pl.BlockSpec((1, a, b), lambda i: (i, 0, 0))
    rowdat, coldat = pl.pallas_call(
        _sort_gather_body,
        grid=(N,),
        in_specs=[
            spec3(1, CAND),
            spec3(CAND, 1),
            spec3(1, CAND),
            spec3(CAND, 1),
            spec3(CAND, 16),
            spec3(16, CAND),
            spec3(1, 8),
        ],
        out_specs=[spec3(8, TOPP), spec3(TOPP, 8)],
        out_shape=[
            jax.ShapeDtypeStruct((N, 8, TOPP), jnp.float32),
            jax.ShapeDtypeStruct((N, TOPP, 8), jnp.float32),
        ],
    )(vals.reshape(N, 1, CAND), vals.reshape(N, CAND, 1),
      idxf.reshape(N, 1, CAND), idxf.reshape(N, CAND, 1),
      rows, rowsT, sizes_f)
    return rowdat, coldat


def _nms_body(rd_ref, cd_ref, bx_ref, sc_ref, lb_ref, adj_ref):
    f32 = jnp.float32
    ts = rd_ref[0, 0:1, :]
    cls_row = rd_ref[0, 1:2, :]
    x1 = rd_ref[0, 2:3, :]
    y1 = rd_ref[0, 3:4, :]
    x2 = rd_ref[0, 4:5, :]
    y2 = rd_ref[0, 5:6, :]

    off_row = cls_row * 10000.0
    bnx1, bny1, bnx2, bny2 = x1 + off_row, y1 + off_row, x2 + off_row, y2 + off_row
    area_row = (bnx2 - bnx1) * (bny2 - bny1)

    det = jnp.sqrt(jnp.clip(ts, 1e-12, None))
    valid = (ts > 0) & ((x2 - x1) >= MIN_SIZE) & ((y2 - y1) >= MIN_SIZE)

    x1c = cd_ref[0, :, 0:1]
    y1c = cd_ref[0, :, 1:2]
    x2c = cd_ref[0, :, 2:3]
    y2c = cd_ref[0, :, 3:4]
    off_col = cd_ref[0, :, 4:5] * 10000.0
    bnx1c, bny1c = x1c + off_col, y1c + off_col
    bnx2c, bny2c = x2c + off_col, y2c + off_col
    area_col = (bnx2c - bnx1c) * (bny2c - bny1c)

    SLAB = 128
    iota_j = lax.broadcasted_iota(jnp.int32, (SLAB, TOPP), 1)
    for s in range(TOPP // SLAB):
        r = slice(s * SLAB, (s + 1) * SLAB)
        xx1 = jnp.maximum(bnx1c[r], bnx1)
        yy1 = jnp.maximum(bny1c[r], bny1)
        xx2 = jnp.minimum(bnx2c[r], bnx2)
        yy2 = jnp.minimum(bny2c[r], bny2)
        iw = jnp.clip(xx2 - xx1, 0.0, None)
        ih = jnp.clip(yy2 - yy1, 0.0, None)
        inter = iw * ih
        iou = inter / jnp.maximum(area_col[r] + area_row - inter, 1e-8)
        tri = iota_j > (s * SLAB + lax.broadcasted_iota(jnp.int32, (SLAB, TOPP), 0))
        adj_ref[r, :] = jnp.where((iou > NMS_THRESH) & tri, 1.0, 0.0)

    lane = lax.broadcasted_iota(jnp.int32, (1, TOPP), 1)
    sup0 = jnp.where(valid, 0.0, 1.0)

    def scan_body(i, sup):
        row = adj_ref[pl.ds(i, 1), :]
        sup_i = jnp.sum(jnp.where(lane == i, sup, 0.0))
        return jnp.maximum(sup, row * (1.0 - jnp.minimum(sup_i, 1.0)))

    sup = lax.fori_loop(0, PRE_NMS_TOP_N, scan_body, sup0)

    final = jnp.where(valid & (sup < 0.5), det, -1.0)

    out_iota = lax.broadcasted_iota(jnp.int32, (1, 128), 1)

    def pick_body(t, carry):
        fin, s_row, l_row, ox1, oy1, ox2, oy2 = carry
        m = jnp.max(fin)
        sel = jnp.min(jnp.where(fin == m, lane, TOPP + 1))
        selm = lane == sel
        ok = m > 0

        def e(v):
            return jnp.sum(jnp.where(selm, v, 0.0))

        tm = out_iota == t
        s_row = jnp.where(tm, jnp.where(ok, m, 0.0), s_row)
        l_row = jnp.where(tm, jnp.where(ok, e(cls_row), 0.0), l_row)
        ox1 = jnp.where(tm, jnp.where(ok, e(x1), 0.0), ox1)
        oy1 = jnp.where(tm, jnp.where(ok, e(y1), 0.0), oy1)
        ox2 = jnp.where(tm, jnp.where(ok, e(x2), 0.0), ox2)
        oy2 = jnp.where(tm, jnp.where(ok, e(y2), 0.0), oy2)
        fin = jnp.where(selm, -2.0, fin)
        return fin, s_row, l_row, ox1, oy1, ox2, oy2

    z = jnp.zeros((1, 128), f32)
    carry = (final, z, z, z, z, z, z)
    _, s_row, l_row, ox1, oy1, ox2, oy2 = lax.fori_loop(
        0, FPN_POST_NMS_TOP_N, pick_body, carry)

    sc_ref[0] = s_row
    lb_ref[0] = l_row.astype(jnp.int32)
    bx_ref[0, 0:1, :] = ox1
    bx_ref[0, 1:2, :] = oy1
    bx_ref[0, 2:3, :] = ox2
    bx_ref[0, 3:4, :] = oy2


def _nms_call(rowdat, coldat):
    spec3 = lambda a, b: pl.BlockSpec((1, a, b), lambda i: (i, 0, 0))
    boxes4, scores, labels = pl.pallas_call(
        _nms_body,
        grid=(N,),
        in_specs=[spec3(8, TOPP), spec3(TOPP, 8)],
        out_specs=[spec3(4, 128), spec3(1, 128), spec3(1, 128)],
        out_shape=[
            jax.ShapeDtypeStruct((N, 4, 128), jnp.float32),
            jax.ShapeDtypeStruct((N, 1, 128), jnp.float32),
            jax.ShapeDtypeStruct((N, 1, 128), jnp.int32),
        ],
        scratch_shapes=[pltpu.VMEM((TOPP, TOPP), jnp.float32)],
    )(rowdat, coldat)
    return boxes4, scores, labels


def kernel(locations, box_cls, box_regression, ang_regression, centerness, image_sizes, is_rotated):
    comb, thr = _comb_scores_bisect(box_cls, centerness)

    regT = box_regression.reshape(N, 4, HW).transpose(0, 2, 1)
    locb = jnp.broadcast_to(locations[None], (N, HW, 2))
    tab = jnp.concatenate(
        [regT, locb, jnp.zeros((N, HW, 122), jnp.float32)], axis=2)
    tab = tab.reshape(N * HW, 128)

    (rows_sc,) = _sc_compact_kernel()(
        comb.reshape(N * CHW), thr[:, 0, :16].reshape(N * 16), tab)
    rows = rows_sc.reshape(N, SCOUT, 128)[:, :CAND, :16]
    vals = rows[:, :, 6]
    idxf = rows[:, :, 7]
    rowsT = rows.transpose(0, 2, 1)

    sizes_f = jnp.pad(image_sizes.astype(jnp.float32), ((0, 0), (0, 6)))
    sizes_f = sizes_f.reshape(N, 1, 8)

    rowdat, coldat = _sort_gather_call(vals, idxf, rows, rowsT, sizes_f)
    boxes4, scores, labels = _nms_call(rowdat, coldat)
    out_boxes = boxes4[:, :, :FPN_POST_NMS_TOP_N].transpose(0, 2, 1)
    out_scores = scores[:, 0, :FPN_POST_NMS_TOP_N]
    out_labels = labels[:, 0, :FPN_POST_NMS_TOP_N]
    return out_boxes, out_scores, out_labels


# speedup vs baseline: 2.8885x; 1.0126x over previous
import functools

import jax
import jax.numpy as jnp
from jax import lax
from jax.experimental import pallas as pl
from jax.experimental.pallas import tpu as pltpu
from jax.experimental.pallas import tpu_sc as plsc

PRE_NMS_THRESH = 0.05
PRE_NMS_TOP_N = 1000
NMS_THRESH = 0.6
FPN_POST_NMS_TOP_N = 100
MIN_SIZE = 0.0
N, C, H, W = 2, 80, 100, 152
HW = H * W
CHW = C * HW
TOPP = 1024
CAND = 1536
SCOUT = 4096
DUMP = 3072
NTILE = 16
PERTILE = CHW // NTILE
ONE_BITS = 0x3F800000

_HI = lax.Precision.HIGHEST


def _comb_bisect_body(cls_ref, ctr_ref, comb_ref, thr_ref):
    s = jax.nn.sigmoid(cls_ref[...])
    c = jax.nn.sigmoid(ctr_ref[...])
    comb = jnp.where(s > PRE_NMS_THRESH, s * c, 0.0)
    comb_ref[...] = comb
    ci = lax.bitcast_convert_type(comb, jnp.int32)

    def body(_, lohi):
        lo, hi = lohi
        mid = (lo + hi) >> 1
        cnt = jnp.sum((ci > mid).astype(jnp.int32))
        pred = cnt >= PRE_NMS_TOP_N
        return jnp.where(pred, mid, lo), jnp.where(pred, hi, mid)

    lo, _ = lax.fori_loop(0, 31, body, (jnp.int32(-1), jnp.int32(ONE_BITS)))
    thr = lax.bitcast_convert_type(jnp.maximum(lo, 0), jnp.float32)
    thr_ref[...] = jnp.broadcast_to(thr, (1, 1, 128))


def _comb_scores_bisect(box_cls, centerness):
    cls3 = box_cls.reshape(N, C, HW)
    ctr3 = centerness.reshape(N, 1, HW)
    return pl.pallas_call(
        _comb_bisect_body,
        grid=(N,),
        in_specs=[
            pl.BlockSpec((1, C, HW), lambda i: (i, 0, 0)),
            pl.BlockSpec((1, 1, HW), lambda i: (i, 0, 0)),
        ],
        out_specs=[
            pl.BlockSpec((1, C, HW), lambda i: (i, 0, 0)),
            pl.BlockSpec((1, 1, 128), lambda i: (i, 0, 0)),
        ],
        out_shape=[
            jax.ShapeDtypeStruct((N, C, HW), jnp.float32),
            jax.ShapeDtypeStruct((N, 1, 128), jnp.float32),
        ],
    )(cls3, ctr3)


def _sc_body(comb_hbm, thr_hbm, tab_hbm, rows_hbm,
             data_v, thr_v, vals_c, idx_c, pos_c, loc_c, rows_v, off_sm,
             sem, sem2):
    img = lax.axis_index("c")
    sub = lax.axis_index("s")
    base = sub * PERTILE
    pltpu.sync_copy(comb_hbm.at[pl.ds(img * CHW + base, PERTILE)], data_v)
    pltpu.sync_copy(thr_hbm.at[pl.ds(img * 16, 16)], thr_v)
    thr = thr_v[...]
    lane = lax.iota(jnp.int32, 16)

    def fill_body(q, _):
        rows_v[0, q >> 3, pl.ds((q & 7) * 16, 16)] = jnp.full((16,), -1.0, jnp.float32)
        return 0

    lax.fori_loop(0, 1024, fill_body, 0)
    seg = SCOUT // NTILE
    pltpu.sync_copy(rows_v.at[0], rows_hbm.at[pl.ds(img * SCOUT + sub * seg, 128)])
    pltpu.sync_copy(rows_v.at[0], rows_hbm.at[pl.ds(img * SCOUT + sub * seg + 128, 128)])

    BLK = 128

    def blk_body(ib, carry):
        cnt, cs = carry
        base_e = ib * BLK
        tv = jnp.zeros((16,), jnp.int32)
        for u in range(BLK // 16):
            v = data_v[pl.ds(base_e + u * 16, 16)]
            tv = tv + jnp.where(v > thr, 1, 0)
        t = jnp.sum(tv)

        @pl.when(t > 0)
        def _():
            c = cnt
            for u in range(BLK // 16):
                v = data_v[pl.ds(base_e + u * 16, 16)]
                m = v > thr
                mi = jnp.where(m, 1, 0)
                pos = c + plsc.cumsum(mi) - mi
                ok = m & (pos < CAND)
                plsc.store_scatter(vals_c, [pos >> 7, pos & 127], v, mask=ok)
                gi = lane + (base_e + u * 16 + base)
                plsc.store_scatter(idx_c, [pos >> 7, pos & 127], gi, mask=ok)
                c = c + jnp.sum(mi)

        return cnt + t, cs + t

    cnt, cs = lax.fori_loop(0, PERTILE // BLK, blk_body,
                            (jnp.zeros((16,), jnp.int32), jnp.int32(0)))

    off_sm[0] = 0
    plsc.subcore_barrier()
    for j in range(NTILE):
        plsc.fetch_and_add(off_sm, jnp.where(j > sub, cs, 0), subcore_id=j)
    plsc.subcore_barrier()
    off = jnp.zeros((16,), jnp.int32) + off_sm[0]

    def pos_body(q, _):
        r = lane + q * 16
        posg = img * SCOUT + jnp.where((r < cnt) & ((off + r) < CAND), off + r, DUMP)
        pos_c[q >> 3, pl.ds((q & 7) * 16, 16)] = posg
        gi = idx_c[q >> 3, pl.ds((q & 7) * 16, 16)]
        loc = gi - (gi // HW) * HW
        loc = jnp.clip(loc, 0, HW - 1) + img * HW
        loc_c[q >> 3, pl.ds((q & 7) * 16, 16)] = loc
        return 0

    lax.fori_loop(0, 128, pos_body, 0)

    for j in range(16):
        b = j % 2

        @pl.when(cs > j * 128)
        def _(j=j, b=b):
            g = pltpu.async_copy(tab_hbm.at[loc_c.at[j]], rows_v.at[b], sem2)
            g.wait()
            for t in range(8):
                rt = lane + t * 16
                v16 = vals_c[j, pl.ds(t * 16, 16)]
                i16 = idx_c[j, pl.ds(t * 16, 16)]
                plsc.store_scatter(rows_v.at[b], [rt, jnp.full((16,), 6, jnp.int32)], v16)
                plsc.store_scatter(rows_v.at[b], [rt, jnp.full((16,), 7, jnp.int32)],
                                   i16.astype(jnp.float32))
            sc = pltpu.async_copy(rows_v.at[b], rows_hbm.at[pos_c.at[j]], sem2)
            sc.wait()


@functools.lru_cache(maxsize=1)
def _sc_compact_kernel():
    @functools.partial(
        pl.kernel,
        mesh=plsc.VectorSubcoreMesh(core_axis_name="c", subcore_axis_name="s"),
        compiler_params=pltpu.CompilerParams(needs_layout_passes=False),
        out_type=[
            jax.ShapeDtypeStruct((N * SCOUT, 128), jnp.float32),
        ],
        scratch_types=[
            pltpu.VMEM((PERTILE,), jnp.float32),
            pltpu.VMEM((16,), jnp.float32),
            pltpu.VMEM((16, 128), jnp.float32),
            pltpu.VMEM((16, 128), jnp.int32),
            pltpu.VMEM((16, 128), jnp.int32),
            pltpu.VMEM((16, 128), jnp.int32),
            pltpu.VMEM((2, 128, 128), jnp.float32),
            pltpu.SMEM((1,), jnp.int32),
            pltpu.SemaphoreType.DMA,
            pltpu.SemaphoreType.DMA,
        ],
    )
    def _sc_compact(comb_hbm, thr_hbm, tab_hbm, rows_hbm, *scratch):
        _sc_body(comb_hbm, thr_hbm, tab_hbm, rows_hbm, *scratch)

    return _sc_compact


def _sort_gather_body(vr_ref, vc_ref, ir_ref, ic_ref, rows_ref, rowsT_ref,
                      sz_ref, rd_ref, cd_ref):
    f32 = jnp.float32
    vrow = vr_ref[0]
    vcol = vc_ref[0]
    irow = ir_ref[0]
    icol = ic_ref[0]
    h_img = sz_ref[0, 0, 0]
    w_img = sz_ref[0, 0, 1]

    ranks = []
    rrow = jnp.zeros((1, CAND), f32)
    for sl in range(CAND // 128):
        vi = vcol[sl * 128:(sl + 1) * 128]
        ii = icol[sl * 128:(sl + 1) * 128]
        cmp = ((vrow > vi) | ((vrow == vi) & (irow < ii))).astype(f32)
        ranks.append(jnp.sum(cmp, axis=1, keepdims=True))
        rrow = rrow + jnp.sum(cmp, axis=0, keepdims=True)
    rank_col = jnp.concatenate(ranks, axis=0)
    rank_row = (CAND - 1.0) - rrow

    PT = (rank_col.astype(jnp.int32)
          == lax.broadcasted_iota(jnp.int32, (CAND, TOPP), 1)).astype(f32)
    P = (rank_row.astype(jnp.int32)
         == lax.broadcasted_iota(jnp.int32, (TOPP, CAND), 0)).astype(f32)
    accT = jnp.dot(rowsT_ref[0], PT, preferred_element_type=f32, precision=_HI)
    accR = jnp.dot(P, rows_ref[0], preferred_element_type=f32, precision=_HI)
    s_val = accT[6:7]
    s_idxf = accT[7:8]
    s_idxf_c = accR[:, 7:8]

    lane = lax.broadcasted_iota(jnp.int32, (1, TOPP), 1)
    ts = jnp.where(lane < PRE_NMS_TOP_N, s_val, -1.0)
    sidx_r = s_idxf.astype(jnp.int32)
    sidx_c = s_idxf_c.astype(jnp.int32)
    cls_row = ((sidx_r // HW) + 1).astype(f32)
    cls_col = ((sidx_c // HW) + 1).astype(f32)

    r0, r1, r2, r3 = (accT[0:1], accT[1:2], accT[2:3], accT[3:4])
    px, py = accT[4:5], accT[5:6]
    x1 = jnp.clip(px - r0, 0.0, w_img - 1.0)
    y1 = jnp.clip(py - r1, 0.0, h_img - 1.0)
    x2 = jnp.clip(px + r2, 0.0, w_img - 1.0)
    y2 = jnp.clip(py + r3, 0.0, h_img - 1.0)
    rd_ref[0, 0:1, :] = ts
    rd_ref[0, 1:2, :] = cls_row
    rd_ref[0, 2:3, :] = x1
    rd_ref[0, 3:4, :] = y1
    rd_ref[0, 4:5, :] = x2
    rd_ref[0, 5:6, :] = y2
    rd_ref[0, 6:7, :] = jnp.zeros((1, TOPP), f32)
    rd_ref[0, 7:8, :] = jnp.zeros((1, TOPP), f32)

    c0, c1, c2, c3 = (accR[:, 0:1], accR[:, 1:2], accR[:, 2:3], accR[:, 3:4])
    pxc, pyc = accR[:, 4:5], accR[:, 5:6]
    x1c = jnp.clip(pxc - c0, 0.0, w_img - 1.0)
    y1c = jnp.clip(pyc - c1, 0.0, h_img - 1.0)
    x2c = jnp.clip(pxc + c2, 0.0, w_img - 1.0)
    y2c = jnp.clip(pyc + c3, 0.0, h_img - 1.0)
    zc = jnp.zeros((TOPP, 1), f32)
    cd_ref[0, :, 0:1] = x1c
    cd_ref[0, :, 1:2] = y1c
    cd_ref[0, :, 2:3] = x2c
    cd_ref[0, :, 3:4] = y2c
    cd_ref[0, :, 4:5] = cls_col
    cd_ref[0, :, 5:6] = zc
    cd_ref[0, :, 6:7] = zc
    cd_ref[0, :, 7:8] = zc


def _sort_gather_call(vals, idxf, rows, rowsT, sizes_f):
    spec3 = lambda a, b: pl.BlockSpec((1, a, b), lambda i: (i, 0, 0))
    rowdat, coldat = pl.pallas_call(
        _sort_gather_body,
        grid=(N,),
        in_specs=[
            spec3(1, CAND),
            spec3(CAND, 1),
            spec3(1, CAND),
            spec3(CAND, 1),
            spec3(CAND, 16),
            spec3(16, CAND),
            spec3(1, 8),
        ],
        out_specs=[spec3(8, TOPP), spec3(TOPP, 8)],
        out_shape=[
            jax.ShapeDtypeStruct((N, 8, TOPP), jnp.float32),
            jax.ShapeDtypeStruct((N, TOPP, 8), jnp.float32),
        ],
    )(vals.reshape(N, 1, CAND), vals.reshape(N, CAND, 1),
      idxf.reshape(N, 1, CAND), idxf.reshape(N, CAND, 1),
      rows, rowsT, sizes_f)
    return rowdat, coldat


def _nms_body(rd_ref, cd_ref, bx_ref, sc_ref, lb_ref, adj_ref):
    f32 = jnp.float32
    ts = rd_ref[0, 0:1, :]
    cls_row = rd_ref[0, 1:2, :]
    x1 = rd_ref[0, 2:3, :]
    y1 = rd_ref[0, 3:4, :]
    x2 = rd_ref[0, 4:5, :]
    y2 = rd_ref[0, 5:6, :]

    off_row = cls_row * 10000.0
    bnx1, bny1, bnx2, bny2 = x1 + off_row, y1 + off_row, x2 + off_row, y2 + off_row
    area_row = (bnx2 - bnx1) * (bny2 - bny1)

    det = jnp.sqrt(jnp.clip(ts, 1e-12, None))
    valid = (ts > 0) & ((x2 - x1) >= MIN_SIZE) & ((y2 - y1) >= MIN_SIZE)

    x1c = cd_ref[0, :, 0:1]
    y1c = cd_ref[0, :, 1:2]
    x2c = cd_ref[0, :, 2:3]
    y2c = cd_ref[0, :, 3:4]
    off_col = cd_ref[0, :, 4:5] * 10000.0
    bnx1c, bny1c = x1c + off_col, y1c + off_col
    bnx2c, bny2c = x2c + off_col, y2c + off_col
    area_col = (bnx2c - bnx1c) * (bny2c - bny1c)

    SLAB = 128
    iota_j = lax.broadcasted_iota(jnp.int32, (SLAB, TOPP), 1)
    for s in range(TOPP // SLAB):
        r = slice(s * SLAB, (s + 1) * SLAB)
        xx1 = jnp.maximum(bnx1c[r], bnx1)
        yy1 = jnp.maximum(bny1c[r], bny1)
        xx2 = jnp.minimum(bnx2c[r], bnx2)
        yy2 = jnp.minimum(bny2c[r], bny2)
        iw = jnp.clip(xx2 - xx1, 0.0, None)
        ih = jnp.clip(yy2 - yy1, 0.0, None)
        inter = iw * ih
        iou = inter / jnp.maximum(area_col[r] + area_row - inter, 1e-8)
        tri = iota_j > (s * SLAB + lax.broadcasted_iota(jnp.int32, (SLAB, TOPP), 0))
        adj_ref[r, :] = jnp.where((iou > NMS_THRESH) & tri, 1.0, 0.0)

    lane = lax.broadcasted_iota(jnp.int32, (1, TOPP), 1)
    sup0 = jnp.where(valid, 0.0, 1.0)

    def scan_body(i, sup):
        row = adj_ref[pl.ds(i, 1), :]
        sup_i = jnp.sum(jnp.where(lane == i, sup, 0.0))
        return jnp.maximum(sup, row * (1.0 - jnp.minimum(sup_i, 1.0)))

    sup = lax.fori_loop(0, PRE_NMS_TOP_N, scan_body, sup0)

    final = jnp.where(valid & (sup < 0.5), det, -1.0)

    out_iota = lax.broadcasted_iota(jnp.int32, (1, 128), 1)

    def pick_body(t, carry):
        fin, s_row, l_row, ox1, oy1, ox2, oy2 = carry
        m = jnp.max(fin)
        sel = jnp.min(jnp.where(fin == m, lane, TOPP + 1))
        selm = lane == sel
        ok = m > 0

        def e(v):
            return jnp.sum(jnp.where(selm, v, 0.0))

        tm = out_iota == t
        s_row = jnp.where(tm, jnp.where(ok, m, 0.0), s_row)
        l_row = jnp.where(tm, jnp.where(ok, e(cls_row), 0.0), l_row)
        ox1 = jnp.where(tm, jnp.where(ok, e(x1), 0.0), ox1)
        oy1 = jnp.where(tm, jnp.where(ok, e(y1), 0.0), oy1)
        ox2 = jnp.where(tm, jnp.where(ok, e(x2), 0.0), ox2)
        oy2 = jnp.where(tm, jnp.where(ok, e(y2), 0.0), oy2)
        fin = jnp.where(selm, -2.0, fin)
        return fin, s_row, l_row, ox1, oy1, ox2, oy2

    z = jnp.zeros((1, 128), f32)
    carry = (final, z, z, z, z, z, z)
    _, s_row, l_row, ox1, oy1, ox2, oy2 = lax.fori_loop(
        0, FPN_POST_NMS_TOP_N, pick_body, carry)

    sc_ref[0] = s_row
    lb_ref[0] = l_row.astype(jnp.int32)
    bx_ref[0, 0:1, :] = ox1
    bx_ref[0, 1:2, :] = oy1
    bx_ref[0, 2:3, :] = ox2
    bx_ref[0, 3:4, :] = oy2


def _nms_call(rowdat, coldat):
    spec3 = lambda a, b: pl.BlockSpec((1, a, b), lambda i: (i, 0, 0))
    boxes4, scores, labels = pl.pallas_call(
        _nms_body,
        grid=(N,),
        in_specs=[spec3(8, TOPP), spec3(TOPP, 8)],
        out_specs=[spec3(4, 128), spec3(1, 128), spec3(1, 128)],
        out_shape=[
            jax.ShapeDtypeStruct((N, 4, 128), jnp.float32),
            jax.ShapeDtypeStruct((N, 1, 128), jnp.float32),
            jax.ShapeDtypeStruct((N, 1, 128), jnp.int32),
        ],
        scratch_shapes=[pltpu.VMEM((TOPP, TOPP), jnp.float32)],
    )(rowdat, coldat)
    return boxes4, scores, labels


def kernel(locations, box_cls, box_regression, ang_regression, centerness, image_sizes, is_rotated):
    comb, thr = _comb_scores_bisect(box_cls, centerness)

    regT = box_regression.reshape(N, 4, HW).transpose(0, 2, 1)
    locb = jnp.broadcast_to(locations[None], (N, HW, 2))
    tab = jnp.concatenate(
        [regT, locb, jnp.zeros((N, HW, 122), jnp.float32)], axis=2)
    tab = tab.reshape(N * HW, 128)

    (rows_sc,) = _sc_compact_kernel()(
        comb.reshape(N * CHW), thr[:, 0, :16].reshape(N * 16), tab)
    rows = rows_sc.reshape(N, SCOUT, 128)[:, :CAND, :16]
    vals = rows[:, :, 6]
    idxf = rows[:, :, 7]
    rowsT = rows.transpose(0, 2, 1)

    sizes_f = jnp.pad(image_sizes.astype(jnp.float32), ((0, 0), (0, 6)))
    sizes_f = sizes_f.reshape(N, 1, 8)

    rowdat, coldat = _sort_gather_call(vals, idxf, rows, rowsT, sizes_f)
    boxes4, scores, labels = _nms_call(rowdat, coldat)
    out_boxes = boxes4[:, :, :FPN_POST_NMS_TOP_N].transpose(0, 2, 1)
    out_scores = scores[:, 0, :FPN_POST_NMS_TOP_N]
    out_labels = labels[:, 0, :FPN_POST_NMS_TOP_N]
    return out_boxes, out_scores, out_labels
